# Initial kernel scaffold; baseline (speedup 1.0000x reference)
import functools

import jax
import jax.numpy as jnp
from jax import lax
from jax.experimental import pallas as pl
from jax.experimental.pallas import tpu as pltpu
from jax.experimental.pallas import tpu_sc as plsc

N = 10000
R = 8
H = 128
L = 16
E = 320000

NC = 2
NS = 16
NW = NC * NS

EP = 323584
G = EP // 128
GPW = G // NW
PAD_BINS = 128
SEGX = R * N + PAD_BINS
SEG_PER_TILE = SEGX // NS
NPT = N // NS

_f32 = jnp.float32
_i32 = jnp.int32


def _mesh():
    return plsc.VectorSubcoreMesh(core_axis_name="c", subcore_axis_name="s",
                                  num_cores=NC, num_subcores=NS)


@functools.partial(
    pl.kernel,
    out_type=jax.ShapeDtypeStruct((NC, SEGX), _f32),
    mesh=_mesh(),
    scratch_types=[
        pltpu.VMEM_SHARED((SEGX,), _f32),
        pltpu.VMEM((1, 128), _i32),
        pltpu.VMEM((128,), _f32),
        pltpu.VMEM((SEG_PER_TILE,), _f32),
    ],
)
def _k1_counts(seg2d, cnt_out, cnt_sh, segbuf, onesbuf, zbuf):
    c = lax.axis_index("c")
    s = lax.axis_index("s")
    w = s * NC + c

    def _z(i, carry):
        zbuf[pl.ds(i * 16, 16)] = jnp.zeros((16,), _f32)
        return carry
    lax.fori_loop(0, SEG_PER_TILE // 16, _z, 0)

    def _o(i, carry):
        onesbuf[pl.ds(i * 16, 16)] = jnp.ones((16,), _f32)
        return carry
    lax.fori_loop(0, 8, _o, 0)

    pltpu.sync_copy(zbuf, cnt_sh.at[pl.ds(s * SEG_PER_TILE, SEG_PER_TILE)])
    plsc.subcore_barrier()

    def _g(g, carry):
        gid = w * GPW + g
        pltpu.sync_copy(seg2d.at[gid], segbuf.at[0])
        pltpu.sync_copy(onesbuf, cnt_sh.at[segbuf.at[0]], add=True)
        return carry
    lax.fori_loop(0, GPW, _g, 0)

    plsc.subcore_barrier()
    pltpu.sync_copy(cnt_sh.at[pl.ds(s * SEG_PER_TILE, SEG_PER_TILE)],
                    cnt_out.at[c, pl.ds(s * SEG_PER_TILE, SEG_PER_TILE)])


def _k1b_body(cnt_ref, invc_ref):
    c = cnt_ref[0] + cnt_ref[1]
    rows = SEGX // 128
    lin = (lax.broadcasted_iota(_i32, (rows, 128), 0) * 128
           + lax.broadcasted_iota(_i32, (rows, 128), 1))
    inv = 1.0 / jnp.maximum(c, 1.0)
    invc_ref[...] = jnp.where(lin < R * N, inv, 0.0)


def _k1b(cnts):
    rows = SEGX // 128
    out = pl.pallas_call(
        _k1b_body,
        out_shape=jax.ShapeDtypeStruct((rows, 128), _f32),
    )(cnts.reshape(NC, rows, 128))
    return out.reshape(SEGX)


@functools.partial(
    pl.kernel,
    out_type=jax.ShapeDtypeStruct((NC, N, 8, 16), _f32),
    mesh=_mesh(),
    scratch_types=[
        pltpu.VMEM_SHARED((N, 8, 16), _f32),
        pltpu.VMEM((1, 128), _i32),
        pltpu.VMEM((1, 128), _i32),
        pltpu.VMEM((1, 128), _i32),
        pltpu.VMEM((128, 8, 16), _f32),
        pltpu.VMEM((128,), _f32),
        pltpu.VMEM((125, 8, 16), _f32),
    ],
)
def _k2_layer1(f12d, seg2d, dst2d, w1f, invc, out_p,
               acc_sh, f1buf, segbuf, dstbuf, rows3, scalebuf, zb):
    c = lax.axis_index("c")
    s = lax.axis_index("s")
    w = s * NC + c

    def _zo(i, carry):
        def _zi(j, carry2):
            zb[i, j] = jnp.zeros((16,), _f32)
            return carry2
        return lax.fori_loop(0, 8, _zi, carry)
    lax.fori_loop(0, 125, _zo, 0)
    for t in range(5):
        pltpu.sync_copy(zb, acc_sh.at[pl.ds(s * NPT + t * 125, 125)])
    plsc.subcore_barrier()

    def _g(g, carry):
        gid = w * GPW + g
        pltpu.sync_copy(f12d.at[gid], f1buf.at[0])
        pltpu.sync_copy(seg2d.at[gid], segbuf.at[0])
        pltpu.sync_copy(dst2d.at[gid], dstbuf.at[0])
        pltpu.sync_copy(w1f.at[f1buf.at[0]], rows3)
        pltpu.sync_copy(invc.at[segbuf.at[0]], scalebuf)

        def _sc(e, carry2):
            sc_ = scalebuf[e]
            for j in range(8):
                rows3[e, j] = rows3[e, j] * sc_
            return carry2
        lax.fori_loop(0, 128, _sc, 0)

        pltpu.sync_copy(rows3, acc_sh.at[dstbuf.at[0]], add=True)
        return carry
    lax.fori_loop(0, GPW, _g, 0)

    plsc.subcore_barrier()
    pltpu.sync_copy(acc_sh.at[pl.ds(s * NPT, NPT)],
                    out_p.at[c, pl.ds(s * NPT, NPT)])


def _k3_body(p0, p1, root1, b1, w2c, r2, hw_ref, rp_ref):
    h = jnp.maximum(p0[...] + p1[...] + root1[...] + b1[...], 0.0)
    hw_ref[...] = jnp.dot(h, w2c[...], preferred_element_type=_f32)
    rp_ref[...] = jnp.dot(h, r2[...], preferred_element_type=_f32)


def _k3(p0, p1, root1, b1, w2cat, root2):
    blk = 1000
    grid = N // blk
    return pl.pallas_call(
        _k3_body,
        grid=(grid,),
        in_specs=[
            pl.BlockSpec((blk, H), lambda i: (i, 0)),
            pl.BlockSpec((blk, H), lambda i: (i, 0)),
            pl.BlockSpec((blk, H), lambda i: (i, 0)),
            pl.BlockSpec((1, H), lambda i: (0, 0)),
            pl.BlockSpec((H, R * L), lambda i: (0, 0)),
            pl.BlockSpec((H, L), lambda i: (0, 0)),
        ],
        out_specs=[
            pl.BlockSpec((blk, R * L), lambda i: (i, 0)),
            pl.BlockSpec((blk, L), lambda i: (i, 0)),
        ],
        out_shape=[
            jax.ShapeDtypeStruct((N, R * L), _f32),
            jax.ShapeDtypeStruct((N, L), _f32),
        ],
    )(p0, p1, root1, b1, w2cat, root2)


@functools.partial(
    pl.kernel,
    out_type=jax.ShapeDtypeStruct((NC, N, L), _f32),
    mesh=_mesh(),
    scratch_types=[
        pltpu.VMEM_SHARED((N, L), _f32),
        pltpu.VMEM((1, 128), _i32),
        pltpu.VMEM((1, 128), _i32),
        pltpu.VMEM((1, 128), _i32),
        pltpu.VMEM((128, L), _f32),
        pltpu.VMEM((128,), _f32),
        pltpu.VMEM((NPT, L), _f32),
    ],
)
def _k4_layer2(f22d, seg2d, dst2d, hwf, invc, out_p,
               acc_sh, f2buf, segbuf, dstbuf, rows2, scalebuf, zb):
    c = lax.axis_index("c")
    s = lax.axis_index("s")
    w = s * NC + c

    def _z(i, carry):
        zb[i] = jnp.zeros((16,), _f32)
        return carry
    lax.fori_loop(0, NPT, _z, 0)
    pltpu.sync_copy(zb, acc_sh.at[pl.ds(s * NPT, NPT)])
    plsc.subcore_barrier()

    def _g(g, carry):
        gid = w * GPW + g
        pltpu.sync_copy(f22d.at[gid], f2buf.at[0])
        pltpu.sync_copy(seg2d.at[gid], segbuf.at[0])
        pltpu.sync_copy(dst2d.at[gid], dstbuf.at[0])
        pltpu.sync_copy(hwf.at[f2buf.at[0]], rows2)
        pltpu.sync_copy(invc.at[segbuf.at[0]], scalebuf)

        def _sc(e, carry2):
            rows2[e] = rows2[e] * scalebuf[e]
            return carry2
        lax.fori_loop(0, 128, _sc, 0)

        pltpu.sync_copy(rows2, acc_sh.at[dstbuf.at[0]], add=True)
        return carry
    lax.fori_loop(0, GPW, _g, 0)

    plsc.subcore_barrier()
    pltpu.sync_copy(acc_sh.at[pl.ds(s * NPT, NPT)],
                    out_p.at[c, pl.ds(s * NPT, NPT)])


def _k5_body(q0, q1, rp, b2, out_ref):
    out_ref[...] = jax.nn.sigmoid(q0[...] + q1[...] + rp[...] + b2[...])


def _k5(q0, q1, rp, b2):
    return pl.pallas_call(
        _k5_body,
        out_shape=jax.ShapeDtypeStruct((N, L), _f32),
    )(q0, q1, rp, b2)


def kernel(edge_index, edge_type, W1, root1, bias1, W2, root2, bias2):
    src = edge_index[0].astype(_i32)
    dst = edge_index[1].astype(_i32)
    et = edge_type.astype(_i32)

    pad = EP - E
    padk = jnp.arange(pad, dtype=_i32) % PAD_BINS
    src_p = jnp.concatenate([src, padk])
    dst_p = jnp.concatenate([dst, padk])
    et_p = jnp.concatenate([et, jnp.full((pad,), R, _i32)])

    etm = et_p & (R - 1)
    seg2d = (et_p * N + dst_p).reshape(G, 128)
    f12d = (etm * N + src_p).reshape(G, 128)
    f22d = (src_p * R + etm).reshape(G, 128)
    dst2d = dst_p.reshape(G, 128)

    cnts = _k1_counts(seg2d)
    invc = _k1b(cnts)

    part1 = _k2_layer1(f12d, seg2d, dst2d,
                       W1.reshape(R * N, 8, 16), invc)

    w2cat = jnp.transpose(W2, (1, 0, 2)).reshape(H, R * L)
    hw, rp = _k3(part1[0].reshape(N, H), part1[1].reshape(N, H),
                 root1, bias1.reshape(1, H), w2cat, root2)

    part2 = _k4_layer2(f22d, seg2d, dst2d,
                       hw.reshape(N * R, L), invc)

    return _k5(part2[0], part2[1], rp, bias2.reshape(1, L))


# speedup vs baseline: 11.3508x; 11.3508x over previous
import functools

import jax
import jax.numpy as jnp
from jax import lax
from jax.experimental import pallas as pl
from jax.experimental.pallas import tpu as pltpu
from jax.experimental.pallas import tpu_sc as plsc

N = 10000
R = 8
H = 128
L = 16
E = 320000

NC = 2
NS = 16
NW = NC * NS

EP = 323584
G = EP // 128
GPW = G // NW
PAD_BINS = 128
SEGX = R * N + PAD_BINS
SEG_PER_TILE = SEGX // NS
NPT = 624
NTAIL = N - NPT * NS

_f32 = jnp.float32
_i32 = jnp.int32


def _mesh():
    return plsc.VectorSubcoreMesh(core_axis_name="c", subcore_axis_name="s",
                                  num_cores=NC, num_subcores=NS)


@functools.partial(
    pl.kernel,
    out_type=jax.ShapeDtypeStruct((NC * SEGX,), _f32),
    mesh=_mesh(),
    scratch_types=[
        pltpu.VMEM_SHARED((SEGX,), _f32),
        pltpu.VMEM((1, 128), _i32),
        pltpu.VMEM((128,), _f32),
        pltpu.VMEM((SEG_PER_TILE,), _f32),
    ],
)
def _k1_counts(seg2d, cnt_out, cnt_sh, segbuf, onesbuf, zbuf):
    c = lax.axis_index("c")
    s = lax.axis_index("s")
    w = s * NC + c

    def _z(i, carry):
        zbuf[pl.ds(i * 16, 16)] = jnp.zeros((16,), _f32)
        return carry
    lax.fori_loop(0, SEG_PER_TILE // 16, _z, 0)

    def _o(i, carry):
        onesbuf[pl.ds(i * 16, 16)] = jnp.ones((16,), _f32)
        return carry
    lax.fori_loop(0, 8, _o, 0)

    pltpu.sync_copy(zbuf, cnt_sh.at[pl.ds(s * SEG_PER_TILE, SEG_PER_TILE)])
    plsc.subcore_barrier()

    def _g(g, carry):
        gid = w * GPW + g
        pltpu.sync_copy(seg2d.at[gid], segbuf.at[0])
        pltpu.sync_copy(onesbuf, cnt_sh.at[segbuf.at[0]], add=True)
        return carry
    lax.fori_loop(0, GPW, _g, 0)

    plsc.subcore_barrier()
    pltpu.sync_copy(cnt_sh.at[pl.ds(s * SEG_PER_TILE, SEG_PER_TILE)], zbuf)
    pltpu.sync_copy(zbuf,
                    cnt_out.at[pl.ds(c * SEGX + s * SEG_PER_TILE,
                                     SEG_PER_TILE)])


def _k1b_body(cnt_ref, invc_ref):
    c = cnt_ref[0] + cnt_ref[1]
    rows = SEGX // 128
    lin = (lax.broadcasted_iota(_i32, (rows, 128), 0) * 128
           + lax.broadcasted_iota(_i32, (rows, 128), 1))
    inv = 1.0 / jnp.maximum(c, 1.0)
    invc_ref[...] = jnp.where(lin < R * N, inv, 0.0)


def _k1b(cnts):
    rows = SEGX // 128
    out = pl.pallas_call(
        _k1b_body,
        out_shape=jax.ShapeDtypeStruct((rows, 128), _f32),
    )(cnts.reshape(NC, rows, 128))
    return out.reshape(SEGX)


@functools.partial(
    pl.kernel,
    out_type=jax.ShapeDtypeStruct((NC, N, H), _f32),
    mesh=_mesh(),
    scratch_types=[
        pltpu.VMEM_SHARED((N, H), _f32),
        pltpu.VMEM((1, 128), _i32),
        pltpu.VMEM((1, 128), _i32),
        pltpu.VMEM((1, 128), _i32),
        pltpu.VMEM((128, H), _f32),
        pltpu.VMEM((128,), _f32),
        pltpu.VMEM((208, H), _f32),
    ],
)
def _k2_layer1(f12d, seg2d, dst2d, w1f, invc, out_p,
               acc_sh, f1buf, segbuf, dstbuf, rows3, scalebuf, zb):
    c = lax.axis_index("c")
    s = lax.axis_index("s")
    w = s * NC + c

    def _zo(i, carry):
        for j in range(8):
            zb[i, pl.ds(j * 16, 16)] = jnp.zeros((16,), _f32)
        return carry
    lax.fori_loop(0, 208, _zo, 0)
    for t in range(3):
        pltpu.sync_copy(zb, acc_sh.at[pl.ds(s * NPT + t * 208, 208)])

    @pl.when(s == NS - 1)
    def _ztail():
        pltpu.sync_copy(zb.at[pl.ds(0, NTAIL)],
                        acc_sh.at[pl.ds(NPT * NS, NTAIL)])
    plsc.subcore_barrier()

    def _g(g, carry):
        gid = w * GPW + g
        pltpu.sync_copy(f12d.at[gid], f1buf.at[0])
        pltpu.sync_copy(seg2d.at[gid], segbuf.at[0])
        pltpu.sync_copy(dst2d.at[gid], dstbuf.at[0])
        pltpu.sync_copy(w1f.at[f1buf.at[0]], rows3)
        pltpu.sync_copy(invc.at[segbuf.at[0]], scalebuf)

        def _sc(k, carry2):
            sv = scalebuf[pl.ds(k * 16, 16)]
            for i in range(16):
                sc_ = sv[i]
                e = k * 16 + i
                for j in range(8):
                    rows3[e, pl.ds(j * 16, 16)] = (
                        rows3[e, pl.ds(j * 16, 16)] * sc_)
            return carry2
        lax.fori_loop(0, 8, _sc, 0)

        pltpu.sync_copy(rows3, acc_sh.at[dstbuf.at[0]], add=True)
        return carry
    lax.fori_loop(0, GPW, _g, 0)

    plsc.subcore_barrier()
    for t in range(3):
        pltpu.sync_copy(acc_sh.at[pl.ds(s * NPT + t * 208, 208)], zb)
        pltpu.sync_copy(zb, out_p.at[c, pl.ds(s * NPT + t * 208, 208)])

    @pl.when(s == NS - 1)
    def _otail():
        pltpu.sync_copy(acc_sh.at[pl.ds(NPT * NS, NTAIL)],
                        zb.at[pl.ds(0, NTAIL)])
        pltpu.sync_copy(zb.at[pl.ds(0, NTAIL)],
                        out_p.at[c, pl.ds(NPT * NS, NTAIL)])


def _k3_body(p0, p1, root1, b1, w2c, r2, hw_ref, rp_ref):
    h = jnp.maximum(p0[...] + p1[...] + root1[...] + b1[...], 0.0)
    hw_ref[...] = jnp.dot(h, w2c[...], preferred_element_type=_f32)
    rp_ref[...] = jnp.dot(h, r2[...], preferred_element_type=_f32)


def _k3(p0, p1, root1, b1, w2cat, root2):
    blk = 1000
    grid = N // blk
    return pl.pallas_call(
        _k3_body,
        grid=(grid,),
        in_specs=[
            pl.BlockSpec((blk, H), lambda i: (i, 0)),
            pl.BlockSpec((blk, H), lambda i: (i, 0)),
            pl.BlockSpec((blk, H), lambda i: (i, 0)),
            pl.BlockSpec((1, H), lambda i: (0, 0)),
            pl.BlockSpec((H, R * L), lambda i: (0, 0)),
            pl.BlockSpec((H, L), lambda i: (0, 0)),
        ],
        out_specs=[
            pl.BlockSpec((blk, R * L), lambda i: (i, 0)),
            pl.BlockSpec((blk, L), lambda i: (i, 0)),
        ],
        out_shape=[
            jax.ShapeDtypeStruct((N, R * L), _f32),
            jax.ShapeDtypeStruct((N, L), _f32),
        ],
    )(p0, p1, root1, b1, w2cat, root2)


@functools.partial(
    pl.kernel,
    out_type=jax.ShapeDtypeStruct((NC, N, L), _f32),
    mesh=_mesh(),
    compiler_params=pltpu.CompilerParams(use_tc_tiling_on_sc=False),
    scratch_types=[
        pltpu.VMEM_SHARED((N, L), _f32),
        pltpu.VMEM((1, 128), _i32),
        pltpu.VMEM((1, 128), _i32),
        pltpu.VMEM((1, 128), _i32),
        pltpu.VMEM((128, L), _f32),
        pltpu.VMEM((128,), _f32),
        pltpu.VMEM((NPT, L), _f32),
    ],
)
def _k4_layer2(f22d, seg2d, dst2d, hwf, invc, out_p,
               acc_sh, f2buf, segbuf, dstbuf, rows2, scalebuf, zb):
    c = lax.axis_index("c")
    s = lax.axis_index("s")
    w = s * NC + c

    def _z(i, carry):
        zb[i] = jnp.zeros((16,), _f32)
        return carry
    lax.fori_loop(0, NPT, _z, 0)
    pltpu.sync_copy(zb, acc_sh.at[pl.ds(s * NPT, NPT)])

    @pl.when(s == NS - 1)
    def _ztail():
        pltpu.sync_copy(zb.at[pl.ds(0, NTAIL)],
                        acc_sh.at[pl.ds(NPT * NS, NTAIL)])
    plsc.subcore_barrier()

    def _g(g, carry):
        gid = w * GPW + g
        pltpu.sync_copy(f22d.at[gid], f2buf.at[0])
        pltpu.sync_copy(seg2d.at[gid], segbuf.at[0])
        pltpu.sync_copy(dst2d.at[gid], dstbuf.at[0])
        pltpu.sync_copy(hwf.at[f2buf.at[0]], rows2)
        pltpu.sync_copy(invc.at[segbuf.at[0]], scalebuf)

        def _sc(k, carry2):
            sv = scalebuf[pl.ds(k * 16, 16)]
            for i in range(16):
                e = k * 16 + i
                rows2[e] = rows2[e] * sv[i]
            return carry2
        lax.fori_loop(0, 8, _sc, 0)

        pltpu.sync_copy(rows2, acc_sh.at[dstbuf.at[0]], add=True)
        return carry
    lax.fori_loop(0, GPW, _g, 0)

    plsc.subcore_barrier()
    pltpu.sync_copy(acc_sh.at[pl.ds(s * NPT, NPT)], zb)
    pltpu.sync_copy(zb, out_p.at[c, pl.ds(s * NPT, NPT)])

    @pl.when(s == NS - 1)
    def _otail():
        pltpu.sync_copy(acc_sh.at[pl.ds(NPT * NS, NTAIL)],
                        zb.at[pl.ds(0, NTAIL)])
        pltpu.sync_copy(zb.at[pl.ds(0, NTAIL)],
                        out_p.at[c, pl.ds(NPT * NS, NTAIL)])


def _k5_body(q0, q1, rp, b2, out_ref):
    out_ref[...] = jax.nn.sigmoid(q0[...] + q1[...] + rp[...] + b2[...])


def _k5(q0, q1, rp, b2):
    return pl.pallas_call(
        _k5_body,
        out_shape=jax.ShapeDtypeStruct((N, L), _f32),
    )(q0, q1, rp, b2)


def kernel(edge_index, edge_type, W1, root1, bias1, W2, root2, bias2):
    src = edge_index[0].astype(_i32)
    dst = edge_index[1].astype(_i32)
    et = edge_type.astype(_i32)

    pad = EP - E
    padk = jnp.arange(pad, dtype=_i32) % PAD_BINS
    src_p = jnp.concatenate([src, padk])
    dst_p = jnp.concatenate([dst, padk])
    et_p = jnp.concatenate([et, jnp.full((pad,), R, _i32)])

    etm = et_p & (R - 1)
    seg2d = (et_p * N + dst_p).reshape(G, 128)
    f12d = (etm * N + src_p).reshape(G, 128)
    f22d = (src_p * R + etm).reshape(G, 128)
    dst2d = dst_p.reshape(G, 128)

    cnts = _k1_counts(seg2d)
    invc = _k1b(cnts)

    part1 = _k2_layer1(f12d, seg2d, dst2d, W1.reshape(R * N, H), invc)

    w2cat = jnp.transpose(W2, (1, 0, 2)).reshape(H, R * L)
    hw, rp = _k3(part1[0], part1[1],
                 root1, bias1.reshape(1, H), w2cat, root2)

    part2 = _k4_layer2(f22d, seg2d, dst2d, hw.reshape(N * R, L), invc)

    return _k5(part2[0], part2[1], rp, bias2.reshape(1, L))


# speedup vs baseline: 12.5329x; 1.1041x over previous
import functools

import jax
import jax.numpy as jnp
from jax import lax
from jax.experimental import pallas as pl
from jax.experimental.pallas import tpu as pltpu
from jax.experimental.pallas import tpu_sc as plsc

N = 10000
R = 8
H = 128
L = 16
E = 320000

NC = 2
NS = 16
NW = NC * NS

EP = 323584
G = EP // 128
GPW = G // NW
PAD_BINS = 128
SEGX = R * N + PAD_BINS
SEG_PER_TILE = SEGX // NS
NPT = 624
NTAIL = N - NPT * NS

_f32 = jnp.float32
_i32 = jnp.int32


def _mesh():
    return plsc.VectorSubcoreMesh(core_axis_name="c", subcore_axis_name="s",
                                  num_cores=NC, num_subcores=NS)


@functools.partial(
    pl.kernel,
    out_type=jax.ShapeDtypeStruct((NC * SEGX,), _f32),
    mesh=_mesh(),
    scratch_types=[
        pltpu.VMEM_SHARED((SEGX,), _f32),
        pltpu.VMEM((1, 128), _i32),
        pltpu.VMEM((128,), _f32),
        pltpu.VMEM((SEG_PER_TILE,), _f32),
    ],
)
def _k1_counts(seg2d, cnt_out, cnt_sh, segbuf, onesbuf, zbuf):
    c = lax.axis_index("c")
    s = lax.axis_index("s")
    w = s * NC + c

    def _z(i, carry):
        zbuf[pl.ds(i * 16, 16)] = jnp.zeros((16,), _f32)
        return carry
    lax.fori_loop(0, SEG_PER_TILE // 16, _z, 0)

    def _o(i, carry):
        onesbuf[pl.ds(i * 16, 16)] = jnp.ones((16,), _f32)
        return carry
    lax.fori_loop(0, 8, _o, 0)

    pltpu.sync_copy(zbuf, cnt_sh.at[pl.ds(s * SEG_PER_TILE, SEG_PER_TILE)])
    plsc.subcore_barrier()

    def _g(g, carry):
        gid = w * GPW + g
        pltpu.sync_copy(seg2d.at[gid], segbuf.at[0])
        pltpu.sync_copy(onesbuf, cnt_sh.at[segbuf.at[0]], add=True)
        return carry
    lax.fori_loop(0, GPW, _g, 0)

    plsc.subcore_barrier()
    pltpu.sync_copy(cnt_sh.at[pl.ds(s * SEG_PER_TILE, SEG_PER_TILE)], zbuf)
    pltpu.sync_copy(zbuf,
                    cnt_out.at[pl.ds(c * SEGX + s * SEG_PER_TILE,
                                     SEG_PER_TILE)])


def _k1b_body(cnt_ref, invc_ref):
    c = cnt_ref[0] + cnt_ref[1]
    rows = SEGX // 128
    lin = (lax.broadcasted_iota(_i32, (rows, 128), 0) * 128
           + lax.broadcasted_iota(_i32, (rows, 128), 1))
    inv = 1.0 / jnp.maximum(c, 1.0)
    invc_ref[...] = jnp.where(lin < R * N, inv, 0.0)


def _k1b(cnts):
    rows = SEGX // 128
    out = pl.pallas_call(
        _k1b_body,
        out_shape=jax.ShapeDtypeStruct((rows, 128), _f32),
    )(cnts.reshape(NC, rows, 128))
    return out.reshape(SEGX)


@functools.partial(
    pl.kernel,
    out_type=jax.ShapeDtypeStruct((NC, N, H), _f32),
    mesh=_mesh(),
    scratch_types=[
        pltpu.VMEM_SHARED((N, H), _f32),
        pltpu.VMEM((3, 128), _i32),
        pltpu.VMEM((128, H), _f32),
        pltpu.VMEM((128,), _f32),
        pltpu.VMEM((208, H), _f32),
    ],
)
def _k2_layer1(idx3, w1f, invc, out_p,
               acc_sh, idxbuf, rows3, scalebuf, zb):
    c = lax.axis_index("c")
    s = lax.axis_index("s")
    w = s * NC + c

    def _zo(i, carry):
        for j in range(8):
            zb[i, pl.ds(j * 16, 16)] = jnp.zeros((16,), _f32)
        return carry
    lax.fori_loop(0, 208, _zo, 0)
    for t in range(3):
        pltpu.sync_copy(zb, acc_sh.at[pl.ds(s * NPT + t * 208, 208)])

    @pl.when(s == NS - 1)
    def _ztail():
        pltpu.sync_copy(zb.at[pl.ds(0, NTAIL)],
                        acc_sh.at[pl.ds(NPT * NS, NTAIL)])
    plsc.subcore_barrier()

    def _g(g, carry):
        gid = w * GPW + g
        pltpu.sync_copy(idx3.at[gid], idxbuf)
        pltpu.sync_copy(w1f.at[idxbuf.at[0]], rows3)
        pltpu.sync_copy(invc.at[idxbuf.at[1]], scalebuf)

        def _sc(k, carry2):
            sv = scalebuf[pl.ds(k * 16, 16)]
            for i in range(16):
                sc_ = sv[i]
                e = k * 16 + i
                for j in range(8):
                    rows3[e, pl.ds(j * 16, 16)] = (
                        rows3[e, pl.ds(j * 16, 16)] * sc_)
            return carry2
        lax.fori_loop(0, 8, _sc, 0)

        pltpu.sync_copy(rows3, acc_sh.at[idxbuf.at[2]], add=True)
        return carry
    lax.fori_loop(0, GPW, _g, 0)

    plsc.subcore_barrier()
    for t in range(3):
        pltpu.sync_copy(acc_sh.at[pl.ds(s * NPT + t * 208, 208)], zb)
        pltpu.sync_copy(zb, out_p.at[c, pl.ds(s * NPT + t * 208, 208)])

    @pl.when(s == NS - 1)
    def _otail():
        pltpu.sync_copy(acc_sh.at[pl.ds(NPT * NS, NTAIL)],
                        zb.at[pl.ds(0, NTAIL)])
        pltpu.sync_copy(zb.at[pl.ds(0, NTAIL)],
                        out_p.at[c, pl.ds(NPT * NS, NTAIL)])


def _k3_body(p0, p1, root1, b1, w2c, r2, hw_ref, rp_ref):
    h = jnp.maximum(p0[...] + p1[...] + root1[...] + b1[...], 0.0)
    hw_ref[...] = jnp.dot(h, w2c[...], preferred_element_type=_f32)
    rp_ref[...] = jnp.dot(h, r2[...], preferred_element_type=_f32)


def _k3(p0, p1, root1, b1, w2cat, root2):
    blk = 1000
    grid = N // blk
    return pl.pallas_call(
        _k3_body,
        grid=(grid,),
        in_specs=[
            pl.BlockSpec((blk, H), lambda i: (i, 0)),
            pl.BlockSpec((blk, H), lambda i: (i, 0)),
            pl.BlockSpec((blk, H), lambda i: (i, 0)),
            pl.BlockSpec((1, H), lambda i: (0, 0)),
            pl.BlockSpec((H, R * L), lambda i: (0, 0)),
            pl.BlockSpec((H, L), lambda i: (0, 0)),
        ],
        out_specs=[
            pl.BlockSpec((blk, R * L), lambda i: (i, 0)),
            pl.BlockSpec((blk, L), lambda i: (i, 0)),
        ],
        out_shape=[
            jax.ShapeDtypeStruct((N, R * L), _f32),
            jax.ShapeDtypeStruct((N, L), _f32),
        ],
    )(p0, p1, root1, b1, w2cat, root2)


SEG_CH = 624
SEG_TAIL = SEG_PER_TILE - 8 * SEG_CH


@functools.partial(
    pl.kernel,
    out_type=jax.ShapeDtypeStruct((NC, SEGX, L), _f32),
    mesh=_mesh(),
    compiler_params=pltpu.CompilerParams(use_tc_tiling_on_sc=False),
    scratch_types=[
        pltpu.VMEM_SHARED((SEGX, L), _f32),
        pltpu.VMEM((2, 128), _i32),
        pltpu.VMEM((128, L), _f32),
        pltpu.VMEM((SEG_CH, L), _f32),
    ],
)
def _k4_layer2(idx2, hwf, out_p, acc_sh, idxbuf, rows2, zb):
    c = lax.axis_index("c")
    s = lax.axis_index("s")
    w = s * NC + c

    def _z(i, carry):
        zb[i] = jnp.zeros((16,), _f32)
        return carry
    lax.fori_loop(0, SEG_CH, _z, 0)
    for t in range(8):
        pltpu.sync_copy(zb, acc_sh.at[pl.ds(s * SEG_PER_TILE + t * SEG_CH,
                                            SEG_CH)])
    pltpu.sync_copy(zb.at[pl.ds(0, SEG_TAIL)],
                    acc_sh.at[pl.ds(s * SEG_PER_TILE + 8 * SEG_CH, SEG_TAIL)])
    plsc.subcore_barrier()

    def _g(g, carry):
        gid = w * GPW + g
        pltpu.sync_copy(idx2.at[gid], idxbuf)
        pltpu.sync_copy(hwf.at[idxbuf.at[0]], rows2)
        pltpu.sync_copy(rows2, acc_sh.at[idxbuf.at[1]], add=True)
        return carry
    lax.fori_loop(0, GPW, _g, 0)

    plsc.subcore_barrier()
    for t in range(8):
        pltpu.sync_copy(acc_sh.at[pl.ds(s * SEG_PER_TILE + t * SEG_CH,
                                        SEG_CH)], zb)
        pltpu.sync_copy(zb, out_p.at[c, pl.ds(s * SEG_PER_TILE + t * SEG_CH,
                                              SEG_CH)])
    pltpu.sync_copy(acc_sh.at[pl.ds(s * SEG_PER_TILE + 8 * SEG_CH, SEG_TAIL)],
                    zb.at[pl.ds(0, SEG_TAIL)])
    pltpu.sync_copy(zb.at[pl.ds(0, SEG_TAIL)],
                    out_p.at[c, pl.ds(s * SEG_PER_TILE + 8 * SEG_CH,
                                      SEG_TAIL)])


def _k5_body(q, invw, rp, b2, out_ref):
    acc = q[0] + q[1]
    agg = (acc * invw[...]).sum(axis=0)
    out_ref[...] = jax.nn.sigmoid(agg + rp[...] + b2[...])


def _k5(q, invw, rp, b2):
    blk = 1000
    grid = N // blk
    return pl.pallas_call(
        _k5_body,
        grid=(grid,),
        in_specs=[
            pl.BlockSpec((NC, R, blk, L), lambda i: (0, 0, i, 0)),
            pl.BlockSpec((R, blk, 1), lambda i: (0, i, 0)),
            pl.BlockSpec((blk, L), lambda i: (i, 0)),
            pl.BlockSpec((1, L), lambda i: (0, 0)),
        ],
        out_specs=pl.BlockSpec((blk, L), lambda i: (i, 0)),
        out_shape=jax.ShapeDtypeStruct((N, L), _f32),
    )(q, invw, rp, b2)


def kernel(edge_index, edge_type, W1, root1, bias1, W2, root2, bias2):
    src = edge_index[0].astype(_i32)
    dst = edge_index[1].astype(_i32)
    et = edge_type.astype(_i32)

    pad = EP - E
    padk = jnp.arange(pad, dtype=_i32) % PAD_BINS
    src_p = jnp.concatenate([src, padk])
    dst_p = jnp.concatenate([dst, padk])
    et_p = jnp.concatenate([et, jnp.full((pad,), R, _i32)])

    etm = et_p & (R - 1)
    seg2d = (et_p * N + dst_p).reshape(G, 128)
    f12d = (etm * N + src_p).reshape(G, 128)
    f22d = (src_p * R + etm).reshape(G, 128)
    dst2d = dst_p.reshape(G, 128)
    idx3 = jnp.stack([f12d, seg2d, dst2d], axis=1)
    idx2 = jnp.stack([f22d, seg2d], axis=1)

    cnts = _k1_counts(seg2d)
    invc = _k1b(cnts)

    part1 = _k2_layer1(idx3, W1.reshape(R * N, H), invc)

    w2cat = jnp.transpose(W2, (1, 0, 2)).reshape(H, R * L)
    hw, rp = _k3(part1[0], part1[1],
                 root1, bias1.reshape(1, H), w2cat, root2)

    part2 = _k4_layer2(idx2, hw.reshape(N * R, L))

    q = part2[:, :R * N, :].reshape(NC, R, N, L)
    invw = invc[:R * N].reshape(R, N, 1)
    return _k5(q, invw, rp, bias2.reshape(1, L))


# speedup vs baseline: 17.7728x; 1.4181x over previous
import functools

import jax
import jax.numpy as jnp
from jax import lax
from jax.experimental import pallas as pl
from jax.experimental.pallas import tpu as pltpu
from jax.experimental.pallas import tpu_sc as plsc

N = 10000
R = 8
H = 128
L = 16
E = 320000

NC = 2
NS = 16
NW = NC * NS

EP = 327680
G = EP // 128
GPW = G // NW
PAD_BINS = 128
SEGX = R * N + PAD_BINS
SEG_PER_TILE = SEGX // NS
NPT = 624
NTAIL = N - NPT * NS

_f32 = jnp.float32
_i32 = jnp.int32


def _mesh():
    return plsc.VectorSubcoreMesh(core_axis_name="c", subcore_axis_name="s",
                                  num_cores=NC, num_subcores=NS)


@functools.partial(
    pl.kernel,
    out_type=jax.ShapeDtypeStruct((NC * SEGX,), _f32),
    mesh=_mesh(),
    scratch_types=[
        pltpu.VMEM_SHARED((SEGX,), _f32),
        pltpu.VMEM((1, 128), _i32),
        pltpu.VMEM((128,), _f32),
        pltpu.VMEM((SEG_PER_TILE,), _f32),
    ],
)
def _k1_counts(seg2d, cnt_out, cnt_sh, segbuf, onesbuf, zbuf):
    c = lax.axis_index("c")
    s = lax.axis_index("s")
    w = s * NC + c

    def _z(i, carry):
        zbuf[pl.ds(i * 16, 16)] = jnp.zeros((16,), _f32)
        return carry
    lax.fori_loop(0, SEG_PER_TILE // 16, _z, 0)

    def _o(i, carry):
        onesbuf[pl.ds(i * 16, 16)] = jnp.ones((16,), _f32)
        return carry
    lax.fori_loop(0, 8, _o, 0)

    pltpu.sync_copy(zbuf, cnt_sh.at[pl.ds(s * SEG_PER_TILE, SEG_PER_TILE)])
    plsc.subcore_barrier()

    def _g(g, carry):
        gid = w * GPW + g
        pltpu.sync_copy(seg2d.at[gid], segbuf.at[0])
        pltpu.sync_copy(onesbuf, cnt_sh.at[segbuf.at[0]], add=True)
        return carry
    lax.fori_loop(0, GPW, _g, 0)

    plsc.subcore_barrier()
    pltpu.sync_copy(cnt_sh.at[pl.ds(s * SEG_PER_TILE, SEG_PER_TILE)], zbuf)
    pltpu.sync_copy(zbuf,
                    cnt_out.at[pl.ds(c * SEGX + s * SEG_PER_TILE,
                                     SEG_PER_TILE)])


def _k1b_body(cnt_ref, invc_ref):
    c = cnt_ref[0] + cnt_ref[1]
    rows = SEGX // 128
    lin = (lax.broadcasted_iota(_i32, (rows, 128), 0) * 128
           + lax.broadcasted_iota(_i32, (rows, 128), 1))
    inv = 1.0 / jnp.maximum(c, 1.0)
    invc_ref[...] = jnp.where(lin < R * N, inv, 0.0)


def _k1b(cnts):
    rows = SEGX // 128
    out = pl.pallas_call(
        _k1b_body,
        out_shape=jax.ShapeDtypeStruct((rows, 128), _f32),
    )(cnts.reshape(NC, rows, 128))
    return out.reshape(SEGX)


@functools.partial(
    pl.kernel,
    out_type=jax.ShapeDtypeStruct((NC, N, H), _f32),
    mesh=_mesh(),
    scratch_types=[
        pltpu.VMEM_SHARED((N, H), _f32),
        pltpu.VMEM((3, 128), _i32),
        pltpu.VMEM((3, 128), _i32),
        pltpu.VMEM((128, H), _f32),
        pltpu.VMEM((128, H), _f32),
        pltpu.VMEM((128,), _f32),
        pltpu.VMEM((128,), _f32),
        pltpu.VMEM((104, H), _f32),
        pltpu.SemaphoreType.DMA,
        pltpu.SemaphoreType.DMA,
        pltpu.SemaphoreType.DMA,
        pltpu.SemaphoreType.DMA,
    ],
)
def _k2_layer1(idx3, w1f, invc, out_p,
               acc_sh, idx0, idx1, rows0, rows1, sc0, sc1, zb,
               gsem0, gsem1, ssem0, ssem1):
    c = lax.axis_index("c")
    s = lax.axis_index("s")
    w = s * NC + c
    idxb = (idx0, idx1)
    rowsb = (rows0, rows1)
    scb = (sc0, sc1)
    gsem = (gsem0, gsem1)
    ssem = (ssem0, ssem1)

    def _zo(i, carry):
        for j in range(8):
            zb[i, pl.ds(j * 16, 16)] = jnp.zeros((16,), _f32)
        return carry
    lax.fori_loop(0, 104, _zo, 0)
    for t in range(6):
        pltpu.sync_copy(zb, acc_sh.at[pl.ds(s * NPT + t * 104, 104)])

    @pl.when(s == NS - 1)
    def _ztail():
        pltpu.sync_copy(zb.at[pl.ds(0, NTAIL)],
                        acc_sh.at[pl.ds(NPT * NS, NTAIL)])
    plsc.subcore_barrier()

    base = w * GPW

    def _fire(b, gid):
        pltpu.sync_copy(idx3.at[gid], idxb[b])
        pltpu.async_copy(w1f.at[idxb[b].at[0]], rowsb[b], gsem[b])
        pltpu.async_copy(invc.at[idxb[b].at[1]], scb[b], gsem[b])

    def _drain_s(b):
        pltpu.make_async_copy(w1f.at[pl.ds(0, 128)], rowsb[b],
                              ssem[b]).wait()

    def _proc(b):
        pltpu.make_async_copy(w1f.at[pl.ds(0, 128)], rowsb[b],
                              gsem[b]).wait()
        pltpu.make_async_copy(invc.at[pl.ds(0, 128)], scb[b],
                              gsem[b]).wait()
        rows = rowsb[b]
        sbuf = scb[b]

        def _sc(k, carry2):
            sv = sbuf[pl.ds(k * 16, 16)]
            for i in range(16):
                sc_ = sv[i]
                e = k * 16 + i
                for j in range(8):
                    rows[e, pl.ds(j * 16, 16)] = (
                        rows[e, pl.ds(j * 16, 16)] * sc_)
            return carry2
        lax.fori_loop(0, 8, _sc, 0)

        pltpu.async_copy(rows, acc_sh.at[idxb[b].at[2]], ssem[b], add=True)

    _fire(0, base)
    _fire(1, base + 1)

    def _lp(j, carry):
        _proc(0)
        _drain_s(0)
        _fire(0, base + 2 * j + 2)
        _proc(1)
        _drain_s(1)
        _fire(1, base + 2 * j + 3)
        return carry
    lax.fori_loop(0, GPW // 2 - 1, _lp, 0)

    _proc(0)
    _proc(1)
    _drain_s(0)
    _drain_s(1)

    plsc.subcore_barrier()
    for t in range(6):
        pltpu.sync_copy(acc_sh.at[pl.ds(s * NPT + t * 104, 104)], zb)
        pltpu.sync_copy(zb, out_p.at[c, pl.ds(s * NPT + t * 104, 104)])

    @pl.when(s == NS - 1)
    def _otail():
        pltpu.sync_copy(acc_sh.at[pl.ds(NPT * NS, NTAIL)],
                        zb.at[pl.ds(0, NTAIL)])
        pltpu.sync_copy(zb.at[pl.ds(0, NTAIL)],
                        out_p.at[c, pl.ds(NPT * NS, NTAIL)])


def _k3_body(p0, p1, root1, b1, w2c, r2, hw_ref, rp_ref):
    h = jnp.maximum(p0[...] + p1[...] + root1[...] + b1[...], 0.0)
    hw_ref[...] = jnp.dot(h, w2c[...], preferred_element_type=_f32)
    rp_ref[...] = jnp.dot(h, r2[...], preferred_element_type=_f32)


def _k3(p0, p1, root1, b1, w2cat, root2):
    blk = 1000
    grid = N // blk
    return pl.pallas_call(
        _k3_body,
        grid=(grid,),
        in_specs=[
            pl.BlockSpec((blk, H), lambda i: (i, 0)),
            pl.BlockSpec((blk, H), lambda i: (i, 0)),
            pl.BlockSpec((blk, H), lambda i: (i, 0)),
            pl.BlockSpec((1, H), lambda i: (0, 0)),
            pl.BlockSpec((H, R * L), lambda i: (0, 0)),
            pl.BlockSpec((H, L), lambda i: (0, 0)),
        ],
        out_specs=[
            pl.BlockSpec((blk, R * L), lambda i: (i, 0)),
            pl.BlockSpec((blk, L), lambda i: (i, 0)),
        ],
        out_shape=[
            jax.ShapeDtypeStruct((N, R * L), _f32),
            jax.ShapeDtypeStruct((N, L), _f32),
        ],
    )(p0, p1, root1, b1, w2cat, root2)


SEG_CH = 624
SEG_TAIL = SEG_PER_TILE - 8 * SEG_CH


@functools.partial(
    pl.kernel,
    out_type=jax.ShapeDtypeStruct((NC, SEGX, L), _f32),
    mesh=_mesh(),
    compiler_params=pltpu.CompilerParams(use_tc_tiling_on_sc=False),
    scratch_types=[
        pltpu.VMEM_SHARED((SEGX, L), _f32),
        pltpu.VMEM((2, 128), _i32),
        pltpu.VMEM((2, 128), _i32),
        pltpu.VMEM((128, L), _f32),
        pltpu.VMEM((128, L), _f32),
        pltpu.VMEM((SEG_CH, L), _f32),
        pltpu.SemaphoreType.DMA,
        pltpu.SemaphoreType.DMA,
        pltpu.SemaphoreType.DMA,
        pltpu.SemaphoreType.DMA,
    ],
)
def _k4_layer2(idx2, hwf, out_p, acc_sh, idx0, idx1, rows0, rows1, zb,
               gsem0, gsem1, ssem0, ssem1):
    c = lax.axis_index("c")
    s = lax.axis_index("s")
    w = s * NC + c
    idxb = (idx0, idx1)
    rowsb = (rows0, rows1)
    gsem = (gsem0, gsem1)
    ssem = (ssem0, ssem1)

    def _z(i, carry):
        zb[i] = jnp.zeros((16,), _f32)
        return carry
    lax.fori_loop(0, SEG_CH, _z, 0)
    for t in range(8):
        pltpu.sync_copy(zb, acc_sh.at[pl.ds(s * SEG_PER_TILE + t * SEG_CH,
                                            SEG_CH)])
    pltpu.sync_copy(zb.at[pl.ds(0, SEG_TAIL)],
                    acc_sh.at[pl.ds(s * SEG_PER_TILE + 8 * SEG_CH, SEG_TAIL)])
    plsc.subcore_barrier()

    base = w * GPW

    def _fire(b, gid):
        pltpu.sync_copy(idx2.at[gid], idxb[b])
        pltpu.async_copy(hwf.at[idxb[b].at[0]], rowsb[b], gsem[b])

    def _drain_s(b):
        pltpu.make_async_copy(hwf.at[pl.ds(0, 128)], rowsb[b],
                              ssem[b]).wait()

    def _proc(b):
        pltpu.make_async_copy(hwf.at[pl.ds(0, 128)], rowsb[b],
                              gsem[b]).wait()
        pltpu.async_copy(rowsb[b], acc_sh.at[idxb[b].at[1]], ssem[b],
                         add=True)

    _fire(0, base)
    _fire(1, base + 1)

    def _lp(j, carry):
        _proc(0)
        _drain_s(0)
        _fire(0, base + 2 * j + 2)
        _proc(1)
        _drain_s(1)
        _fire(1, base + 2 * j + 3)
        return carry
    lax.fori_loop(0, GPW // 2 - 1, _lp, 0)

    _proc(0)
    _proc(1)
    _drain_s(0)
    _drain_s(1)

    plsc.subcore_barrier()
    for t in range(8):
        pltpu.sync_copy(acc_sh.at[pl.ds(s * SEG_PER_TILE + t * SEG_CH,
                                        SEG_CH)], zb)
        pltpu.sync_copy(zb, out_p.at[c, pl.ds(s * SEG_PER_TILE + t * SEG_CH,
                                              SEG_CH)])
    pltpu.sync_copy(acc_sh.at[pl.ds(s * SEG_PER_TILE + 8 * SEG_CH, SEG_TAIL)],
                    zb.at[pl.ds(0, SEG_TAIL)])
    pltpu.sync_copy(zb.at[pl.ds(0, SEG_TAIL)],
                    out_p.at[c, pl.ds(s * SEG_PER_TILE + 8 * SEG_CH,
                                      SEG_TAIL)])


def _k5_body(q, invw, rp, b2, out_ref):
    acc = q[0] + q[1]
    agg = (acc * invw[...]).sum(axis=0)
    out_ref[...] = jax.nn.sigmoid(agg + rp[...] + b2[...])


def _k5(q, invw, rp, b2):
    blk = 1000
    grid = N // blk
    return pl.pallas_call(
        _k5_body,
        grid=(grid,),
        in_specs=[
            pl.BlockSpec((NC, R, blk, L), lambda i: (0, 0, i, 0)),
            pl.BlockSpec((R, blk, 1), lambda i: (0, i, 0)),
            pl.BlockSpec((blk, L), lambda i: (i, 0)),
            pl.BlockSpec((1, L), lambda i: (0, 0)),
        ],
        out_specs=pl.BlockSpec((blk, L), lambda i: (i, 0)),
        out_shape=jax.ShapeDtypeStruct((N, L), _f32),
    )(q, invw, rp, b2)


def kernel(edge_index, edge_type, W1, root1, bias1, W2, root2, bias2):
    src = edge_index[0].astype(_i32)
    dst = edge_index[1].astype(_i32)
    et = edge_type.astype(_i32)

    pad = EP - E
    padk = jnp.arange(pad, dtype=_i32) % PAD_BINS
    src_p = jnp.concatenate([src, padk])
    dst_p = jnp.concatenate([dst, padk])
    et_p = jnp.concatenate([et, jnp.full((pad,), R, _i32)])

    etm = et_p & (R - 1)
    seg2d = (et_p * N + dst_p).reshape(G, 128)
    f12d = (etm * N + src_p).reshape(G, 128)
    f22d = (src_p * R + etm).reshape(G, 128)
    dst2d = dst_p.reshape(G, 128)
    idx3 = jnp.stack([f12d, seg2d, dst2d], axis=1)
    idx2 = jnp.stack([f22d, seg2d], axis=1)

    cnts = _k1_counts(seg2d)
    invc = _k1b(cnts)

    part1 = _k2_layer1(idx3, W1.reshape(R * N, H), invc)

    w2cat = jnp.transpose(W2, (1, 0, 2)).reshape(H, R * L)
    hw, rp = _k3(part1[0], part1[1],
                 root1, bias1.reshape(1, H), w2cat, root2)

    part2 = _k4_layer2(idx2, hw.reshape(N * R, L))

    q = part2[:, :R * N, :].reshape(NC, R, N, L)
    invw = invc[:R * N].reshape(R, N, 1)
    return _k5(q, invw, rp, bias2.reshape(1, L))


# speedup vs baseline: 23.9349x; 1.3467x over previous
import functools

import jax
import jax.numpy as jnp
from jax import lax
from jax.experimental import pallas as pl
from jax.experimental.pallas import tpu as pltpu
from jax.experimental.pallas import tpu_sc as plsc

N = 10000
R = 8
H = 128
L = 16
E = 320000

NC = 2
NS = 16
NW = NC * NS

EP = 327680
G = EP // 128
GPW = G // NW
PAD_BINS = 128
SEGX = R * N + PAD_BINS
SEG_PER_TILE = SEGX // NS
NPT = 624
NTAIL = N - NPT * NS

_f32 = jnp.float32
_i32 = jnp.int32


def _mesh():
    return plsc.VectorSubcoreMesh(core_axis_name="c", subcore_axis_name="s",
                                  num_cores=NC, num_subcores=NS)


@functools.partial(
    pl.kernel,
    out_type=jax.ShapeDtypeStruct((NC * SEGX,), _f32),
    mesh=_mesh(),
    scratch_types=[
        pltpu.VMEM_SHARED((SEGX,), _f32),
        pltpu.VMEM((1, 128), _i32),
        pltpu.VMEM((128,), _f32),
        pltpu.VMEM((SEG_PER_TILE,), _f32),
    ],
)
def _k1_counts(seg2d, cnt_out, cnt_sh, segbuf, onesbuf, zbuf):
    c = lax.axis_index("c")
    s = lax.axis_index("s")
    w = s * NC + c

    def _z(i, carry):
        zbuf[pl.ds(i * 16, 16)] = jnp.zeros((16,), _f32)
        return carry
    lax.fori_loop(0, SEG_PER_TILE // 16, _z, 0)

    def _o(i, carry):
        onesbuf[pl.ds(i * 16, 16)] = jnp.ones((16,), _f32)
        return carry
    lax.fori_loop(0, 8, _o, 0)

    pltpu.sync_copy(zbuf, cnt_sh.at[pl.ds(s * SEG_PER_TILE, SEG_PER_TILE)])
    plsc.subcore_barrier()

    def _g(g, carry):
        gid = w * GPW + g
        pltpu.sync_copy(seg2d.at[gid], segbuf.at[0])
        pltpu.sync_copy(onesbuf, cnt_sh.at[segbuf.at[0]], add=True)
        return carry
    lax.fori_loop(0, GPW, _g, 0)

    plsc.subcore_barrier()
    pltpu.sync_copy(cnt_sh.at[pl.ds(s * SEG_PER_TILE, SEG_PER_TILE)], zbuf)
    pltpu.sync_copy(zbuf,
                    cnt_out.at[pl.ds(c * SEGX + s * SEG_PER_TILE,
                                     SEG_PER_TILE)])


def _k1b_body(cnt_ref, invc_ref):
    c = cnt_ref[0] + cnt_ref[1]
    rows = SEGX // 128
    lin = (lax.broadcasted_iota(_i32, (rows, 128), 0) * 128
           + lax.broadcasted_iota(_i32, (rows, 128), 1))
    inv = 1.0 / jnp.maximum(c, 1.0)
    invc_ref[...] = jnp.where(lin < R * N, inv, 0.0)


def _k1b(cnts):
    rows = SEGX // 128
    out = pl.pallas_call(
        _k1b_body,
        out_shape=jax.ShapeDtypeStruct((rows, 128), _f32),
    )(cnts.reshape(NC, rows, 128))
    return out.reshape(SEGX)


@functools.partial(
    pl.kernel,
    out_type=jax.ShapeDtypeStruct((NC, N, H), _f32),
    mesh=_mesh(),
    scratch_types=[
        pltpu.VMEM_SHARED((N, H), _f32),
        pltpu.VMEM((3, 128), _i32),
        pltpu.VMEM((3, 128), _i32),
        pltpu.VMEM((128, H), _f32),
        pltpu.VMEM((128, H), _f32),
        pltpu.VMEM((128,), _f32),
        pltpu.VMEM((128,), _f32),
        pltpu.VMEM((104, H), _f32),
        pltpu.SemaphoreType.DMA,
        pltpu.SemaphoreType.DMA,
        pltpu.SemaphoreType.DMA,
        pltpu.SemaphoreType.DMA,
    ],
)
def _k2_layer1(idx3, w1f, invc, out_p,
               acc_sh, idx0, idx1, rows0, rows1, sc0, sc1, zb,
               gsem0, gsem1, ssem0, ssem1):
    c = lax.axis_index("c")
    s = lax.axis_index("s")
    w = s * NC + c
    idxb = (idx0, idx1)
    rowsb = (rows0, rows1)
    scb = (sc0, sc1)
    gsem = (gsem0, gsem1)
    ssem = (ssem0, ssem1)

    def _zo(i, carry):
        for j in range(8):
            zb[i, pl.ds(j * 16, 16)] = jnp.zeros((16,), _f32)
        return carry
    lax.fori_loop(0, 104, _zo, 0)
    for t in range(6):
        pltpu.sync_copy(zb, acc_sh.at[pl.ds(s * NPT + t * 104, 104)])

    @pl.when(s == NS - 1)
    def _ztail():
        pltpu.sync_copy(zb.at[pl.ds(0, NTAIL)],
                        acc_sh.at[pl.ds(NPT * NS, NTAIL)])
    plsc.subcore_barrier()

    base = w * GPW

    def _fire(b, gid):
        pltpu.sync_copy(idx3.at[gid], idxb[b])
        pltpu.async_copy(w1f.at[idxb[b].at[0]], rowsb[b], gsem[b])
        pltpu.async_copy(invc.at[idxb[b].at[1]], scb[b], gsem[b])

    def _drain_s(b):
        pltpu.make_async_copy(w1f.at[pl.ds(0, 128)], rowsb[b],
                              ssem[b]).wait()

    def _proc(b):
        pltpu.make_async_copy(w1f.at[pl.ds(0, 128)], rowsb[b],
                              gsem[b]).wait()
        pltpu.make_async_copy(invc.at[pl.ds(0, 128)], scb[b],
                              gsem[b]).wait()
        rows = rowsb[b]
        sbuf = scb[b]

        def _sc(k, carry2):
            sv = sbuf[pl.ds(k * 16, 16)]
            for i in range(16):
                sc_ = sv[i]
                e = k * 16 + i
                for j in range(8):
                    rows[e, pl.ds(j * 16, 16)] = (
                        rows[e, pl.ds(j * 16, 16)] * sc_)
            return carry2
        lax.fori_loop(0, 8, _sc, 0)

        pltpu.async_copy(rows, acc_sh.at[idxb[b].at[2]], ssem[b], add=True)

    _fire(0, base)
    _fire(1, base + 1)

    def _lp(j, carry):
        _proc(0)
        _drain_s(0)
        _fire(0, base + 2 * j + 2)
        _proc(1)
        _drain_s(1)
        _fire(1, base + 2 * j + 3)
        return carry
    lax.fori_loop(0, GPW // 2 - 1, _lp, 0)

    _proc(0)
    _proc(1)
    _drain_s(0)
    _drain_s(1)

    plsc.subcore_barrier()
    for t in range(6):
        pltpu.sync_copy(acc_sh.at[pl.ds(s * NPT + t * 104, 104)], zb)
        pltpu.sync_copy(zb, out_p.at[c, pl.ds(s * NPT + t * 104, 104)])

    @pl.when(s == NS - 1)
    def _otail():
        pltpu.sync_copy(acc_sh.at[pl.ds(NPT * NS, NTAIL)],
                        zb.at[pl.ds(0, NTAIL)])
        pltpu.sync_copy(zb.at[pl.ds(0, NTAIL)],
                        out_p.at[c, pl.ds(NPT * NS, NTAIL)])


def _k3_body(p0, p1, root1, b1, w2c, r2, hw_ref, rp_ref):
    h = jnp.maximum(p0[...] + p1[...] + root1[...] + b1[...], 0.0)
    hw_ref[...] = jnp.dot(h, w2c[...], preferred_element_type=_f32)
    rp_ref[...] = jnp.dot(h, r2[...], preferred_element_type=_f32)


def _k3(p0, p1, root1, b1, w2cat, root2):
    blk = 1000
    grid = N // blk
    return pl.pallas_call(
        _k3_body,
        grid=(grid,),
        in_specs=[
            pl.BlockSpec((blk, H), lambda i: (i, 0)),
            pl.BlockSpec((blk, H), lambda i: (i, 0)),
            pl.BlockSpec((blk, H), lambda i: (i, 0)),
            pl.BlockSpec((1, H), lambda i: (0, 0)),
            pl.BlockSpec((H, R * L), lambda i: (0, 0)),
            pl.BlockSpec((H, L), lambda i: (0, 0)),
        ],
        out_specs=[
            pl.BlockSpec((blk, R * L), lambda i: (i, 0)),
            pl.BlockSpec((blk, L), lambda i: (i, 0)),
        ],
        out_shape=[
            jax.ShapeDtypeStruct((N, R * L), _f32),
            jax.ShapeDtypeStruct((N, L), _f32),
        ],
    )(p0, p1, root1, b1, w2cat, root2)


SEG_CH = 208
NZCH = SEG_PER_TILE // SEG_CH
SEG_TAIL = SEG_PER_TILE - NZCH * SEG_CH


@functools.partial(
    pl.kernel,
    out_type=jax.ShapeDtypeStruct((NC, N, R * L), _f32),
    mesh=_mesh(),
    compiler_params=pltpu.CompilerParams(use_tc_tiling_on_sc=False),
    scratch_types=[
        pltpu.VMEM_SHARED((SEGX, L), _f32),
        pltpu.VMEM((2, 128), _i32),
        pltpu.VMEM((2, 128), _i32),
        pltpu.VMEM((128, L), _f32),
        pltpu.VMEM((128, L), _f32),
        pltpu.VMEM((R, SEG_CH, L), _f32),
        pltpu.SemaphoreType.DMA,
        pltpu.SemaphoreType.DMA,
        pltpu.SemaphoreType.DMA,
        pltpu.SemaphoreType.DMA,
        pltpu.SemaphoreType.DMA,
    ],
)
def _k4_layer2(idx2, hwf, out_p, acc_sh, idx0, idx1, rows0, rows1, zbr,
               gsem0, gsem1, ssem0, ssem1, dsem):
    c = lax.axis_index("c")
    s = lax.axis_index("s")
    w = s * NC + c
    idxb = (idx0, idx1)
    rowsb = (rows0, rows1)
    gsem = (gsem0, gsem1)
    ssem = (ssem0, ssem1)

    def _z(i, carry):
        zbr[0, i] = jnp.zeros((16,), _f32)
        return carry
    lax.fori_loop(0, SEG_CH, _z, 0)
    for t in range(NZCH):
        pltpu.sync_copy(zbr.at[0],
                        acc_sh.at[pl.ds(s * SEG_PER_TILE + t * SEG_CH,
                                        SEG_CH)])
    pltpu.sync_copy(zbr.at[0, pl.ds(0, SEG_TAIL)],
                    acc_sh.at[pl.ds(s * SEG_PER_TILE + NZCH * SEG_CH,
                                    SEG_TAIL)])
    plsc.subcore_barrier()

    base = w * GPW

    def _fire(b, gid):
        pltpu.sync_copy(idx2.at[gid], idxb[b])
        pltpu.async_copy(hwf.at[idxb[b].at[0]], rowsb[b], gsem[b])

    def _drain_s(b):
        pltpu.make_async_copy(hwf.at[pl.ds(0, 128)], rowsb[b],
                              ssem[b]).wait()

    def _proc(b):
        pltpu.make_async_copy(hwf.at[pl.ds(0, 128)], rowsb[b],
                              gsem[b]).wait()
        pltpu.async_copy(rowsb[b], acc_sh.at[idxb[b].at[1]], ssem[b],
                         add=True)

    _fire(0, base)
    _fire(1, base + 1)

    def _lp(j, carry):
        _proc(0)
        _drain_s(0)
        _fire(0, base + 2 * j + 2)
        _proc(1)
        _drain_s(1)
        _fire(1, base + 2 * j + 3)
        return carry
    lax.fori_loop(0, GPW // 2 - 1, _lp, 0)

    _proc(0)
    _proc(1)
    _drain_s(0)
    _drain_s(1)

    plsc.subcore_barrier()

    def _dump(noff, nlen):
        for r in range(R):
            pltpu.sync_copy(acc_sh.at[pl.ds(r * N + noff, nlen)],
                            zbr.at[r, pl.ds(0, nlen)])
        for r in range(R):
            pltpu.async_copy(zbr.at[r, pl.ds(0, nlen)],
                             out_p.at[c, pl.ds(noff, nlen),
                                      pl.ds(r * L, L)], dsem)
        for r in range(R):
            pltpu.make_async_copy(hwf.at[pl.ds(0, nlen)],
                                  zbr.at[r, pl.ds(0, nlen)], dsem).wait()

    for t in range(3):
        _dump(s * NPT + t * SEG_CH, SEG_CH)

    @pl.when(s == NS - 1)
    def _otail():
        _dump(NPT * NS, NTAIL)


def _k5_body(q, scm, rp, b2, out_ref):
    acc = (q[0] + q[1]) * scm[...]
    tot = acc[:, 0:L]
    for r in range(1, R):
        tot = tot + acc[:, r * L:(r + 1) * L]
    out_ref[...] = jax.nn.sigmoid(tot + rp[...] + b2[...])


def _k5(q, scm, rp, b2):
    blk = 1000
    grid = N // blk
    return pl.pallas_call(
        _k5_body,
        grid=(grid,),
        in_specs=[
            pl.BlockSpec((NC, blk, R * L), lambda i: (0, i, 0)),
            pl.BlockSpec((blk, R * L), lambda i: (i, 0)),
            pl.BlockSpec((blk, L), lambda i: (i, 0)),
            pl.BlockSpec((1, L), lambda i: (0, 0)),
        ],
        out_specs=pl.BlockSpec((blk, L), lambda i: (i, 0)),
        out_shape=jax.ShapeDtypeStruct((N, L), _f32),
    )(q, scm, rp, b2)


def kernel(edge_index, edge_type, W1, root1, bias1, W2, root2, bias2):
    src = edge_index[0].astype(_i32)
    dst = edge_index[1].astype(_i32)
    et = edge_type.astype(_i32)

    pad = EP - E
    padk = jnp.arange(pad, dtype=_i32) % PAD_BINS
    src_p = jnp.concatenate([src, padk])
    dst_p = jnp.concatenate([dst, padk])
    et_p = jnp.concatenate([et, jnp.full((pad,), R, _i32)])

    etm = et_p & (R - 1)
    seg2d = (et_p * N + dst_p).reshape(G, 128)
    f12d = (etm * N + src_p).reshape(G, 128)
    f22d = (src_p * R + etm).reshape(G, 128)
    dst2d = dst_p.reshape(G, 128)
    idx3 = jnp.stack([f12d, seg2d, dst2d], axis=1)
    idx2 = jnp.stack([f22d, seg2d], axis=1)

    cnts = _k1_counts(seg2d)
    invc = _k1b(cnts)

    part1 = _k2_layer1(idx3, W1.reshape(R * N, H), invc)

    w2cat = jnp.transpose(W2, (1, 0, 2)).reshape(H, R * L)
    hw, rp = _k3(part1[0], part1[1],
                 root1, bias1.reshape(1, H), w2cat, root2)

    part2 = _k4_layer2(idx2, hw.reshape(N * R, L))

    scmat = jnp.repeat(invc[:R * N].reshape(R, N).T, L, axis=1)
    return _k5(part2, scmat, rp, bias2.reshape(1, L))


# speedup vs baseline: 25.4703x; 1.0641x over previous
import functools

import jax
import jax.numpy as jnp
from jax import lax
from jax.experimental import pallas as pl
from jax.experimental.pallas import tpu as pltpu
from jax.experimental.pallas import tpu_sc as plsc

N = 10000
R = 8
H = 128
L = 16
E = 320000

NC = 2
NS = 16
NW = NC * NS

EP = 327680
G = EP // 128
GPW = G // NW
PAD_BINS = 128
SEGX = R * N + PAD_BINS
SEG_PER_TILE = SEGX // NS
NPT = 624
NTAIL = N - NPT * NS

_f32 = jnp.float32
_i32 = jnp.int32


def _mesh():
    return plsc.VectorSubcoreMesh(core_axis_name="c", subcore_axis_name="s",
                                  num_cores=NC, num_subcores=NS)


@functools.partial(
    pl.kernel,
    out_type=jax.ShapeDtypeStruct((NC * SEGX,), _f32),
    mesh=_mesh(),
    scratch_types=[
        pltpu.VMEM_SHARED((SEGX,), _f32),
        pltpu.VMEM((1, 128), _i32),
        pltpu.VMEM((1, 128), _i32),
        pltpu.VMEM((128,), _f32),
        pltpu.VMEM((SEG_PER_TILE,), _f32),
        pltpu.SemaphoreType.DMA,
        pltpu.SemaphoreType.DMA,
        pltpu.SemaphoreType.DMA,
        pltpu.SemaphoreType.DMA,
    ],
)
def _k1_counts(seg2d, cnt_out, cnt_sh, seg0, seg1, onesbuf, zbuf,
               lsem0, lsem1, ssem0, ssem1):
    c = lax.axis_index("c")
    s = lax.axis_index("s")
    w = s * NC + c
    segb = (seg0, seg1)
    lsem = (lsem0, lsem1)
    ssem = (ssem0, ssem1)

    def _z(i, carry):
        zbuf[pl.ds(i * 16, 16)] = jnp.zeros((16,), _f32)
        return carry
    lax.fori_loop(0, SEG_PER_TILE // 16, _z, 0)

    def _o(i, carry):
        onesbuf[pl.ds(i * 16, 16)] = jnp.ones((16,), _f32)
        return carry
    lax.fori_loop(0, 8, _o, 0)

    pltpu.sync_copy(zbuf, cnt_sh.at[pl.ds(s * SEG_PER_TILE, SEG_PER_TILE)])
    plsc.subcore_barrier()

    base = w * GPW

    def _fire(b, gid):
        pltpu.async_copy(seg2d.at[gid], segb[b].at[0], lsem[b])

    def _proc(b):
        pltpu.make_async_copy(seg2d.at[pl.ds(0, 1)], segb[b],
                              lsem[b]).wait()
        pltpu.async_copy(onesbuf, cnt_sh.at[segb[b].at[0]], ssem[b],
                         add=True)

    def _drain_s(b):
        pltpu.make_async_copy(cnt_out.at[pl.ds(0, 128)], onesbuf,
                              ssem[b]).wait()

    _fire(0, base)
    _fire(1, base + 1)

    def _lp(j, carry):
        _proc(0)
        _drain_s(0)
        _fire(0, base + 2 * j + 2)
        _proc(1)
        _drain_s(1)
        _fire(1, base + 2 * j + 3)
        return carry
    lax.fori_loop(0, GPW // 2 - 1, _lp, 0)

    _proc(0)
    _proc(1)
    _drain_s(0)
    _drain_s(1)

    plsc.subcore_barrier()
    pltpu.sync_copy(cnt_sh.at[pl.ds(s * SEG_PER_TILE, SEG_PER_TILE)], zbuf)
    pltpu.sync_copy(zbuf,
                    cnt_out.at[pl.ds(c * SEGX + s * SEG_PER_TILE,
                                     SEG_PER_TILE)])


def _k1b_body(cnt_ref, invc_ref):
    c = cnt_ref[0] + cnt_ref[1]
    rows = SEGX // 128
    lin = (lax.broadcasted_iota(_i32, (rows, 128), 0) * 128
           + lax.broadcasted_iota(_i32, (rows, 128), 1))
    inv = 1.0 / jnp.maximum(c, 1.0)
    invc_ref[...] = jnp.where(lin < R * N, inv, 0.0)


def _k1b(cnts):
    rows = SEGX // 128
    out = pl.pallas_call(
        _k1b_body,
        out_shape=jax.ShapeDtypeStruct((rows, 128), _f32),
    )(cnts.reshape(NC, rows, 128))
    return out.reshape(SEGX)


@functools.partial(
    pl.kernel,
    out_type=jax.ShapeDtypeStruct((NC, N, H), _f32),
    mesh=_mesh(),
    scratch_types=[
        pltpu.VMEM_SHARED((N, H), _f32),
        pltpu.VMEM((3, 128), _i32),
        pltpu.VMEM((3, 128), _i32),
        pltpu.VMEM((128, H), _f32),
        pltpu.VMEM((128, H), _f32),
        pltpu.VMEM((128,), _f32),
        pltpu.VMEM((128,), _f32),
        pltpu.VMEM((104, H), _f32),
        pltpu.SemaphoreType.DMA,
        pltpu.SemaphoreType.DMA,
        pltpu.SemaphoreType.DMA,
        pltpu.SemaphoreType.DMA,
    ],
)
def _k2_layer1(idx3, w1f, invc, out_p,
               acc_sh, idx0, idx1, rows0, rows1, sc0, sc1, zb,
               gsem0, gsem1, ssem0, ssem1):
    c = lax.axis_index("c")
    s = lax.axis_index("s")
    w = s * NC + c
    idxb = (idx0, idx1)
    rowsb = (rows0, rows1)
    scb = (sc0, sc1)
    gsem = (gsem0, gsem1)
    ssem = (ssem0, ssem1)

    def _zo(i, carry):
        for j in range(8):
            zb[i, pl.ds(j * 16, 16)] = jnp.zeros((16,), _f32)
        return carry
    lax.fori_loop(0, 104, _zo, 0)
    for t in range(6):
        pltpu.sync_copy(zb, acc_sh.at[pl.ds(s * NPT + t * 104, 104)])

    @pl.when(s == NS - 1)
    def _ztail():
        pltpu.sync_copy(zb.at[pl.ds(0, NTAIL)],
                        acc_sh.at[pl.ds(NPT * NS, NTAIL)])
    plsc.subcore_barrier()

    base = w * GPW

    def _fire(b, gid):
        pltpu.sync_copy(idx3.at[gid], idxb[b])
        pltpu.async_copy(w1f.at[idxb[b].at[0]], rowsb[b], gsem[b])
        pltpu.async_copy(invc.at[idxb[b].at[1]], scb[b], gsem[b])

    def _drain_s(b):
        pltpu.make_async_copy(w1f.at[pl.ds(0, 128)], rowsb[b],
                              ssem[b]).wait()

    def _proc(b):
        pltpu.make_async_copy(w1f.at[pl.ds(0, 128)], rowsb[b],
                              gsem[b]).wait()
        pltpu.make_async_copy(invc.at[pl.ds(0, 128)], scb[b],
                              gsem[b]).wait()
        rows = rowsb[b]
        sbuf = scb[b]

        def _sc(k, carry2):
            sv = sbuf[pl.ds(k * 16, 16)]
            for i in range(16):
                sc_ = sv[i]
                e = k * 16 + i
                for j in range(8):
                    rows[e, pl.ds(j * 16, 16)] = (
                        rows[e, pl.ds(j * 16, 16)] * sc_)
            return carry2
        lax.fori_loop(0, 8, _sc, 0)

        pltpu.async_copy(rows, acc_sh.at[idxb[b].at[2]], ssem[b], add=True)

    _fire(0, base)
    _fire(1, base + 1)

    def _lp(j, carry):
        _proc(0)
        _drain_s(0)
        _fire(0, base + 2 * j + 2)
        _proc(1)
        _drain_s(1)
        _fire(1, base + 2 * j + 3)
        return carry
    lax.fori_loop(0, GPW // 2 - 1, _lp, 0)

    _proc(0)
    _proc(1)
    _drain_s(0)
    _drain_s(1)

    plsc.subcore_barrier()
    for t in range(6):
        pltpu.sync_copy(acc_sh.at[pl.ds(s * NPT + t * 104, 104)], zb)
        pltpu.sync_copy(zb, out_p.at[c, pl.ds(s * NPT + t * 104, 104)])

    @pl.when(s == NS - 1)
    def _otail():
        pltpu.sync_copy(acc_sh.at[pl.ds(NPT * NS, NTAIL)],
                        zb.at[pl.ds(0, NTAIL)])
        pltpu.sync_copy(zb.at[pl.ds(0, NTAIL)],
                        out_p.at[c, pl.ds(NPT * NS, NTAIL)])


def _k3_body(p0, p1, root1, b1, w2c, r2, hw_ref, rp_ref):
    h = jnp.maximum(p0[...] + p1[...] + root1[...] + b1[...], 0.0)
    hw_ref[...] = jnp.dot(h, w2c[...], preferred_element_type=_f32)
    rp_ref[...] = jnp.dot(h, r2[...], preferred_element_type=_f32)


def _k3(p0, p1, root1, b1, w2cat, root2):
    blk = 1000
    grid = N // blk
    return pl.pallas_call(
        _k3_body,
        grid=(grid,),
        in_specs=[
            pl.BlockSpec((blk, H), lambda i: (i, 0)),
            pl.BlockSpec((blk, H), lambda i: (i, 0)),
            pl.BlockSpec((blk, H), lambda i: (i, 0)),
            pl.BlockSpec((1, H), lambda i: (0, 0)),
            pl.BlockSpec((H, R * L), lambda i: (0, 0)),
            pl.BlockSpec((H, L), lambda i: (0, 0)),
        ],
        out_specs=[
            pl.BlockSpec((blk, R * L), lambda i: (i, 0)),
            pl.BlockSpec((blk, L), lambda i: (i, 0)),
        ],
        out_shape=[
            jax.ShapeDtypeStruct((N, R * L), _f32),
            jax.ShapeDtypeStruct((N, L), _f32),
        ],
    )(p0, p1, root1, b1, w2cat, root2)


SEG_CH = 208
NZCH = SEG_PER_TILE // SEG_CH
SEG_TAIL = SEG_PER_TILE - NZCH * SEG_CH


@functools.partial(
    pl.kernel,
    out_type=jax.ShapeDtypeStruct((NC, N, R * L), _f32),
    mesh=_mesh(),
    compiler_params=pltpu.CompilerParams(use_tc_tiling_on_sc=False),
    scratch_types=[
        pltpu.VMEM_SHARED((SEGX, L), _f32),
        pltpu.VMEM((2, 128), _i32),
        pltpu.VMEM((2, 128), _i32),
        pltpu.VMEM((128, L), _f32),
        pltpu.VMEM((128, L), _f32),
        pltpu.VMEM((R, SEG_CH, L), _f32),
        pltpu.SemaphoreType.DMA,
        pltpu.SemaphoreType.DMA,
        pltpu.SemaphoreType.DMA,
        pltpu.SemaphoreType.DMA,
        pltpu.SemaphoreType.DMA,
    ],
)
def _k4_layer2(idx2, hwf, out_p, acc_sh, idx0, idx1, rows0, rows1, zbr,
               gsem0, gsem1, ssem0, ssem1, dsem):
    c = lax.axis_index("c")
    s = lax.axis_index("s")
    w = s * NC + c
    idxb = (idx0, idx1)
    rowsb = (rows0, rows1)
    gsem = (gsem0, gsem1)
    ssem = (ssem0, ssem1)

    def _z(i, carry):
        zbr[0, i] = jnp.zeros((16,), _f32)
        return carry
    lax.fori_loop(0, SEG_CH, _z, 0)
    for t in range(NZCH):
        pltpu.sync_copy(zbr.at[0],
                        acc_sh.at[pl.ds(s * SEG_PER_TILE + t * SEG_CH,
                                        SEG_CH)])
    pltpu.sync_copy(zbr.at[0, pl.ds(0, SEG_TAIL)],
                    acc_sh.at[pl.ds(s * SEG_PER_TILE + NZCH * SEG_CH,
                                    SEG_TAIL)])
    plsc.subcore_barrier()

    base = w * GPW

    def _fire(b, gid):
        pltpu.sync_copy(idx2.at[gid], idxb[b])
        pltpu.async_copy(hwf.at[idxb[b].at[0]], rowsb[b], gsem[b])

    def _drain_s(b):
        pltpu.make_async_copy(hwf.at[pl.ds(0, 128)], rowsb[b],
                              ssem[b]).wait()

    def _proc(b):
        pltpu.make_async_copy(hwf.at[pl.ds(0, 128)], rowsb[b],
                              gsem[b]).wait()
        pltpu.async_copy(rowsb[b], acc_sh.at[idxb[b].at[1]], ssem[b],
                         add=True)

    _fire(0, base)
    _fire(1, base + 1)

    def _lp(j, carry):
        _proc(0)
        _drain_s(0)
        _fire(0, base + 2 * j + 2)
        _proc(1)
        _drain_s(1)
        _fire(1, base + 2 * j + 3)
        return carry
    lax.fori_loop(0, GPW // 2 - 1, _lp, 0)

    _proc(0)
    _proc(1)
    _drain_s(0)
    _drain_s(1)

    plsc.subcore_barrier()

    def _dump(noff, nlen):
        for r in range(R):
            pltpu.sync_copy(acc_sh.at[pl.ds(r * N + noff, nlen)],
                            zbr.at[r, pl.ds(0, nlen)])
        for r in range(R):
            pltpu.async_copy(zbr.at[r, pl.ds(0, nlen)],
                             out_p.at[c, pl.ds(noff, nlen),
                                      pl.ds(r * L, L)], dsem)
        for r in range(R):
            pltpu.make_async_copy(hwf.at[pl.ds(0, nlen)],
                                  zbr.at[r, pl.ds(0, nlen)], dsem).wait()

    for t in range(3):
        _dump(s * NPT + t * SEG_CH, SEG_CH)

    @pl.when(s == NS - 1)
    def _otail():
        _dump(NPT * NS, NTAIL)


def _k5_body(q, scm, rp, b2, out_ref):
    acc = (q[0] + q[1]) * scm[...]
    tot = acc[:, 0:L]
    for r in range(1, R):
        tot = tot + acc[:, r * L:(r + 1) * L]
    out_ref[...] = jax.nn.sigmoid(tot + rp[...] + b2[...])


def _k5(q, scm, rp, b2):
    blk = 1000
    grid = N // blk
    return pl.pallas_call(
        _k5_body,
        grid=(grid,),
        in_specs=[
            pl.BlockSpec((NC, blk, R * L), lambda i: (0, i, 0)),
            pl.BlockSpec((blk, R * L), lambda i: (i, 0)),
            pl.BlockSpec((blk, L), lambda i: (i, 0)),
            pl.BlockSpec((1, L), lambda i: (0, 0)),
        ],
        out_specs=pl.BlockSpec((blk, L), lambda i: (i, 0)),
        out_shape=jax.ShapeDtypeStruct((N, L), _f32),
    )(q, scm, rp, b2)


def kernel(edge_index, edge_type, W1, root1, bias1, W2, root2, bias2):
    src = edge_index[0].astype(_i32)
    dst = edge_index[1].astype(_i32)
    et = edge_type.astype(_i32)

    pad = EP - E
    padk = jnp.arange(pad, dtype=_i32) % PAD_BINS
    src_p = jnp.concatenate([src, padk])
    dst_p = jnp.concatenate([dst, padk])
    et_p = jnp.concatenate([et, jnp.full((pad,), R, _i32)])

    etm = et_p & (R - 1)
    seg2d = (et_p * N + dst_p).reshape(G, 128)
    f12d = (etm * N + src_p).reshape(G, 128)
    f22d = (src_p * R + etm).reshape(G, 128)
    dst2d = dst_p.reshape(G, 128)
    idx3 = jnp.stack([f12d, seg2d, dst2d], axis=1)
    idx2 = jnp.stack([f22d, seg2d], axis=1)

    cnts = _k1_counts(seg2d)
    invc = _k1b(cnts)

    part1 = _k2_layer1(idx3, W1.reshape(R * N, H), invc)

    w2cat = jnp.transpose(W2, (1, 0, 2)).reshape(H, R * L)
    hw, rp = _k3(part1[0], part1[1],
                 root1, bias1.reshape(1, H), w2cat, root2)

    part2 = _k4_layer2(idx2, hw.reshape(N * R, L))

    scmat = jnp.repeat(invc[:R * N].reshape(R, N).T, L, axis=1)
    return _k5(part2, scmat, rp, bias2.reshape(1, L))
